# baseline (speedup 1.0000x reference)
import functools

import jax
import jax.numpy as jnp
from jax import lax
from jax.experimental import pallas as pl
from jax.experimental.pallas import tpu as pltpu
from jax.experimental.pallas import tpu_sc as plsc

_NEG_POS = 3.0
_H = 512
_W = 512
_N = float(_H * _W)
_PLANES = 64
_SC_NC = 2
_SC_NS = 16


def _sl1_of_mag(x):
    return jnp.where(x < 1.0, 0.5 * x * x, x - 0.5)


def _field_block(fields, shape, axis):
    ii = lax.broadcasted_iota(jnp.int32, shape, axis)
    out = jnp.zeros(shape, jnp.float32)
    for r, f in enumerate(fields):
        out = out + jnp.where(ii == r, f, 0.0)
    return out


def _dense_body(yt_ref, yp_ref, acc_ref, rz_ref, rp_ref, st_ref):
    p = pl.program_id(0)

    yt = yt_ref[0, 0]
    yp = yp_ref[0, 0]
    ad = jnp.abs(yt - yp)
    sl1 = _sl1_of_mag(ad)
    posb = yt >= 0.5
    posf = posb.astype(jnp.float32)
    z = jnp.logical_or(posb, ad == 0.0).astype(jnp.float32)
    psl1 = sl1 * posf

    rz = jnp.sum(z, axis=1, keepdims=True)
    rp = jnp.sum(psl1, axis=1, keepdims=True)
    num_pos = jnp.sum(posf)
    sl1_tot = jnp.sum(sl1)
    pos_sl1 = jnp.sum(rp)
    S_nz = sl1_tot - pos_sl1
    nz = _N - jnp.sum(rz)
    k = jnp.minimum(_NEG_POS * num_pos, _N - 1.0)
    needA = jnp.logical_and(k <= nz, k > 0.0).astype(jnp.float32)

    rz_ref[0] = rz
    rp_ref[0] = rp
    st_ref[0] = _field_block([num_pos, nz, S_nz, k], (1, 128), 1)

    @pl.when(p == 0)
    def _():
        acc_ref[...] = jnp.zeros_like(acc_ref)

    acc_ref[...] += _field_block([num_pos, k, pos_sl1, needA], (8, 128), 0)


def _dense_call(y_true, y_pred):
    B, C, H, W = y_true.shape
    spec = pl.BlockSpec((1, 1, H, W), lambda p: (p // 4, p % 4, 0, 0))
    return pl.pallas_call(
        _dense_body,
        grid=(_PLANES,),
        in_specs=[spec, spec],
        out_specs=[
            pl.BlockSpec((8, 128), lambda p: (0, 0)),
            pl.BlockSpec((1, H, 1), lambda p: (p, 0, 0)),
            pl.BlockSpec((1, H, 1), lambda p: (p, 0, 0)),
            pl.BlockSpec((1, 1, 128), lambda p: (p, 0, 0)),
        ],
        out_shape=[
            jax.ShapeDtypeStruct((8, 128), jnp.float32),
            jax.ShapeDtypeStruct((_PLANES, H, 1), jnp.float32),
            jax.ShapeDtypeStruct((_PLANES, H, 1), jnp.float32),
            jax.ShapeDtypeStruct((_PLANES, 1, 128), jnp.float32),
        ],
        compiler_params=pltpu.CompilerParams(
            dimension_semantics=("arbitrary",)),
    )(y_true, y_pred)


def _sc_finish_body(rz_hbm, rp_hbm, st_hbm, yt_hbm, yp_hbm, out_hbm,
                    rzv, rpv, rowt, rowp, stv, outv):
    cid = lax.axis_index("c")
    sid = lax.axis_index("s")
    wid = sid * _SC_NC + cid

    def do_plane(j, carry):
        p = wid * 2 + j
        b = p // 4
        ch = p % 4
        pltpu.sync_copy(st_hbm.at[p], stv)
        pltpu.sync_copy(rz_hbm.at[p], rzv)
        pltpu.sync_copy(rp_hbm.at[p], rpv)
        sv = stv[pl.ds(0, 16)]
        nz = sv[1]
        S_nz = sv[2]
        k = sv[3]
        m = k - nz

        def scan_rows(i, c2):
            cum, rstar, fullsum, ro_r = c2
            v = rzv[pl.ds(i * 16, 16)]
            rpc = rpv[pl.ds(i * 16, 16)]
            inc = plsc.cumsum(v) + cum
            fm = inc <= m
            rstar = rstar + jnp.sum(jnp.where(fm, 1.0, 0.0))
            fullsum = fullsum + jnp.sum(jnp.where(fm, rpc, 0.0))
            ro_r = ro_r + jnp.sum(jnp.where(fm, v, 0.0))
            return jnp.max(inc), rstar, fullsum, ro_r

        _, rstar, fullsum, ro_r = lax.fori_loop(
            0, 32, scan_rows, (0.0, 0.0, 0.0, 0.0))
        r_i = jnp.clip(rstar.astype(jnp.int32), 0, _H - 1)

        pltpu.sync_copy(yt_hbm.at[b, ch, r_i], rowt)
        pltpu.sync_copy(yp_hbm.at[b, ch, r_i], rowp)

        def scan_row(i, c2):
            cumz, partial = c2
            t = rowt[pl.ds(i * 16, 16)]
            q = rowp[pl.ds(i * 16, 16)]
            adv = jnp.abs(t - q)
            posv = t >= 0.5
            zv = jnp.where(jnp.logical_or(posv, adv == 0.0), 1.0, 0.0)
            incz = plsc.cumsum(zv) + cumz
            excl = incz - zv
            sl1v = jnp.where(adv < 1.0, 0.5 * adv * adv, adv - 0.5)
            sel = jnp.logical_and(posv, (ro_r + excl) < m)
            partial = partial + jnp.sum(jnp.where(sel, sl1v, 0.0))
            return jnp.max(incz), partial

        _, partial = lax.fori_loop(0, 32, scan_row, (0.0, 0.0))

        negsum = jnp.where(
            k > 0.0, S_nz + jnp.where(k > nz, fullsum + partial, 0.0), 0.0)
        li = lax.broadcasted_iota(jnp.int32, (16,), 0)
        outv[...] = jnp.where(li == 0, negsum, 0.0)
        pltpu.sync_copy(outv, out_hbm.at[p])
        return carry

    lax.fori_loop(0, _PLANES // (_SC_NC * _SC_NS), do_plane, 0)


def _sc_finish(rz3, rp3, st3, y_true, y_pred):
    mesh = plsc.VectorSubcoreMesh(
        core_axis_name="c", subcore_axis_name="s",
        num_cores=_SC_NC, num_subcores=_SC_NS)
    fn = pl.kernel(
        _sc_finish_body,
        out_type=jax.ShapeDtypeStruct((_PLANES, 16), jnp.float32),
        mesh=mesh,
        compiler_params=pltpu.CompilerParams(needs_layout_passes=False),
        scratch_types=[
            pltpu.VMEM((_H,), jnp.float32),
            pltpu.VMEM((_H,), jnp.float32),
            pltpu.VMEM((_W,), jnp.float32),
            pltpu.VMEM((_W,), jnp.float32),
            pltpu.VMEM((128,), jnp.float32),
            pltpu.VMEM((16,), jnp.float32),
        ],
    )
    return fn(rz3.reshape(_PLANES, _H), rp3.reshape(_PLANES, _H),
              st3.reshape(_PLANES, 128), y_true, y_pred)


def _fallback_body(yt_ref, yp_ref, acc_ref):
    p = pl.program_id(0)

    yt = yt_ref[0, 0]
    yp = yp_ref[0, 0]
    ad = jnp.abs(yt - yp)
    sl1 = _sl1_of_mag(ad)
    posf = (yt >= 0.5).astype(jnp.float32)
    negf = 1.0 - posf
    loss = ad * negf
    floss = sl1 * negf

    num_pos = jnp.sum(posf)
    nz = jnp.sum((loss > 0.0).astype(jnp.float32))
    S_nz = jnp.sum(floss)
    k = jnp.minimum(_NEG_POS * num_pos, _N - 1.0)
    needA = jnp.logical_and(k <= nz, k > 0.0)

    bits = lax.bitcast_convert_type(loss, jnp.int32)

    def body(i, lo):
        cand = lo | (1 << (30 - i)).astype(jnp.int32)
        cnt = jnp.sum((bits >= cand).astype(jnp.float32))
        return jnp.where(cnt >= k, cand, lo)

    tbits = lax.fori_loop(0, 31, body, jnp.int32(0))
    t = lax.bitcast_convert_type(tbits, jnp.float32)
    gt = (bits > tbits).astype(jnp.float32)
    cnt_gt = jnp.sum(gt)
    sum_gt = jnp.sum(floss * gt)
    negA = sum_gt + (k - cnt_gt) * _sl1_of_mag(t)
    delta = jnp.where(needA, negA - S_nz, 0.0)

    @pl.when(p == 0)
    def _():
        acc_ref[...] = jnp.zeros_like(acc_ref)

    acc_ref[...] += _field_block([delta], (8, 128), 0)


def _fallback_call(y_true, y_pred):
    B, C, H, W = y_true.shape
    spec = pl.BlockSpec((1, 1, H, W), lambda p: (p // 4, p % 4, 0, 0))
    return pl.pallas_call(
        _fallback_body,
        grid=(_PLANES,),
        in_specs=[spec, spec],
        out_specs=pl.BlockSpec((8, 128), lambda p: (0, 0)),
        out_shape=jax.ShapeDtypeStruct((8, 128), jnp.float32),
        compiler_params=pltpu.CompilerParams(
            dimension_semantics=("arbitrary",)),
    )(y_true, y_pred)


@jax.jit
def kernel(y_true, y_pred):
    acc, rz3, rp3, st3 = _dense_call(y_true, y_pred)
    neg_rows = _sc_finish(rz3, rp3, st3, y_true, y_pred)
    negB = jnp.sum(neg_rows[:, 0])
    pos_cnt = jnp.maximum(acc[0, 0], 1.0)
    neg_cnt = jnp.maximum(acc[1, 0], 1.0)
    delta = lax.cond(
        acc[3, 0] > 0.5,
        lambda: _fallback_call(y_true, y_pred)[0, 0],
        lambda: jnp.float32(0.0),
    )
    return _NEG_POS * acc[2, 0] / pos_cnt + (negB + delta) / neg_cnt


# speedup vs baseline: 1.2693x; 1.2693x over previous
import functools

import jax
import jax.numpy as jnp
from jax import lax
from jax.experimental import pallas as pl
from jax.experimental.pallas import tpu as pltpu
from jax.experimental.pallas import tpu_sc as plsc

_NEG_POS = 3.0
_H = 512
_W = 512
_N = float(_H * _W)
_PLANES = 64
_SC_NC = 2
_SC_NS = 16


def _sl1_of_mag(x):
    return jnp.where(x < 1.0, 0.5 * x * x, x - 0.5)


def _field_block(fields, shape, axis):
    ii = lax.broadcasted_iota(jnp.int32, shape, axis)
    out = jnp.zeros(shape, jnp.float32)
    for r, f in enumerate(fields):
        out = out + jnp.where(ii == r, f, 0.0)
    return out


_CP = 4


def _dense_body(yt_ref, yp_ref, acc_ref, rz_ref, rp_ref, st_ref):
    g = pl.program_id(0)

    yt = yt_ref[0].reshape(_CP * _H, _W)
    yp = yp_ref[0].reshape(_CP * _H, _W)
    ad = jnp.abs(yt - yp)
    sl1 = _sl1_of_mag(ad)
    posb = yt >= 0.5
    posf = posb.astype(jnp.float32)
    z = jnp.logical_or(posb, ad == 0.0).astype(jnp.float32)
    psl1 = sl1 * posf

    rz = jnp.sum(z, axis=1, keepdims=True)
    rp = jnp.sum(psl1, axis=1, keepdims=True)
    rz_ref[...] = rz.reshape(_CP, _H, 1)
    rp_ref[...] = rp.reshape(_CP, _H, 1)

    tot = jnp.zeros((8, 128), jnp.float32)
    for ci in range(_CP):
        sl = slice(ci * _H, (ci + 1) * _H)
        num_pos = jnp.sum(posf[sl])
        sl1_tot = jnp.sum(sl1[sl])
        pos_sl1 = jnp.sum(rp[sl])
        S_nz = sl1_tot - pos_sl1
        nz = _N - jnp.sum(rz[sl])
        k = jnp.minimum(_NEG_POS * num_pos, _N - 1.0)
        needA = jnp.logical_and(k <= nz, k > 0.0).astype(jnp.float32)
        st_ref[ci] = _field_block([num_pos, nz, S_nz, k], (1, 128), 1)
        tot = tot + _field_block([num_pos, k, pos_sl1, needA], (8, 128), 0)

    @pl.when(g == 0)
    def _():
        acc_ref[...] = jnp.zeros_like(acc_ref)

    acc_ref[...] += tot


def _dense_call(y_true, y_pred):
    B, C, H, W = y_true.shape
    spec = pl.BlockSpec((1, _CP, H, W), lambda g: (g, 0, 0, 0))
    return pl.pallas_call(
        _dense_body,
        grid=(_PLANES // _CP,),
        in_specs=[spec, spec],
        out_specs=[
            pl.BlockSpec((8, 128), lambda g: (0, 0)),
            pl.BlockSpec((_CP, H, 1), lambda g: (g, 0, 0)),
            pl.BlockSpec((_CP, H, 1), lambda g: (g, 0, 0)),
            pl.BlockSpec((_CP, 1, 128), lambda g: (g, 0, 0)),
        ],
        out_shape=[
            jax.ShapeDtypeStruct((8, 128), jnp.float32),
            jax.ShapeDtypeStruct((_PLANES, H, 1), jnp.float32),
            jax.ShapeDtypeStruct((_PLANES, H, 1), jnp.float32),
            jax.ShapeDtypeStruct((_PLANES, 1, 128), jnp.float32),
        ],
        compiler_params=pltpu.CompilerParams(
            dimension_semantics=("arbitrary",)),
    )(y_true, y_pred)


def _sc_finish_body(rz_hbm, rp_hbm, st_hbm, yt_hbm, yp_hbm, out_hbm,
                    rzv, rpv, rowt, rowp, stv, outv):
    cid = lax.axis_index("c")
    sid = lax.axis_index("s")
    wid = sid * _SC_NC + cid

    def do_plane(j, carry):
        p = wid * 2 + j
        b = p // 4
        ch = p % 4
        pltpu.sync_copy(st_hbm.at[p], stv)
        pltpu.sync_copy(rz_hbm.at[p], rzv)
        pltpu.sync_copy(rp_hbm.at[p], rpv)
        sv = stv[pl.ds(0, 16)]
        nz = sv[1]
        S_nz = sv[2]
        k = sv[3]
        m = k - nz

        def scan_rows(i, c2):
            cum, rstar, fullsum, ro_r = c2
            v = rzv[pl.ds(i * 16, 16)]
            rpc = rpv[pl.ds(i * 16, 16)]
            inc = plsc.cumsum(v) + cum
            fm = inc <= m
            rstar = rstar + jnp.sum(jnp.where(fm, 1.0, 0.0))
            fullsum = fullsum + jnp.sum(jnp.where(fm, rpc, 0.0))
            ro_r = ro_r + jnp.sum(jnp.where(fm, v, 0.0))
            return jnp.max(inc), rstar, fullsum, ro_r

        _, rstar, fullsum, ro_r = lax.fori_loop(
            0, 32, scan_rows, (0.0, 0.0, 0.0, 0.0))
        r_i = jnp.clip(rstar.astype(jnp.int32), 0, _H - 1)

        pltpu.sync_copy(yt_hbm.at[b, ch, r_i], rowt)
        pltpu.sync_copy(yp_hbm.at[b, ch, r_i], rowp)

        def scan_row(i, c2):
            cumz, partial = c2
            t = rowt[pl.ds(i * 16, 16)]
            q = rowp[pl.ds(i * 16, 16)]
            adv = jnp.abs(t - q)
            posv = t >= 0.5
            zv = jnp.where(jnp.logical_or(posv, adv == 0.0), 1.0, 0.0)
            incz = plsc.cumsum(zv) + cumz
            excl = incz - zv
            sl1v = jnp.where(adv < 1.0, 0.5 * adv * adv, adv - 0.5)
            sel = jnp.logical_and(posv, (ro_r + excl) < m)
            partial = partial + jnp.sum(jnp.where(sel, sl1v, 0.0))
            return jnp.max(incz), partial

        _, partial = lax.fori_loop(0, 32, scan_row, (0.0, 0.0))

        negsum = jnp.where(
            k > 0.0, S_nz + jnp.where(k > nz, fullsum + partial, 0.0), 0.0)
        li = lax.broadcasted_iota(jnp.int32, (16,), 0)
        outv[...] = jnp.where(li == 0, negsum, 0.0)
        pltpu.sync_copy(outv, out_hbm.at[p])
        return carry

    lax.fori_loop(0, _PLANES // (_SC_NC * _SC_NS), do_plane, 0)


def _sc_finish(rz3, rp3, st3, y_true, y_pred):
    mesh = plsc.VectorSubcoreMesh(
        core_axis_name="c", subcore_axis_name="s",
        num_cores=_SC_NC, num_subcores=_SC_NS)
    fn = pl.kernel(
        _sc_finish_body,
        out_type=jax.ShapeDtypeStruct((_PLANES, 16), jnp.float32),
        mesh=mesh,
        compiler_params=pltpu.CompilerParams(needs_layout_passes=False),
        scratch_types=[
            pltpu.VMEM((_H,), jnp.float32),
            pltpu.VMEM((_H,), jnp.float32),
            pltpu.VMEM((_W,), jnp.float32),
            pltpu.VMEM((_W,), jnp.float32),
            pltpu.VMEM((128,), jnp.float32),
            pltpu.VMEM((16,), jnp.float32),
        ],
    )
    return fn(rz3.reshape(_PLANES, _H), rp3.reshape(_PLANES, _H),
              st3.reshape(_PLANES, 128), y_true, y_pred)


def _fallback_body(yt_ref, yp_ref, acc_ref):
    p = pl.program_id(0)

    yt = yt_ref[0, 0]
    yp = yp_ref[0, 0]
    ad = jnp.abs(yt - yp)
    sl1 = _sl1_of_mag(ad)
    posf = (yt >= 0.5).astype(jnp.float32)
    negf = 1.0 - posf
    loss = ad * negf
    floss = sl1 * negf

    num_pos = jnp.sum(posf)
    nz = jnp.sum((loss > 0.0).astype(jnp.float32))
    S_nz = jnp.sum(floss)
    k = jnp.minimum(_NEG_POS * num_pos, _N - 1.0)
    needA = jnp.logical_and(k <= nz, k > 0.0)

    bits = lax.bitcast_convert_type(loss, jnp.int32)

    def body(i, lo):
        cand = lo | (1 << (30 - i)).astype(jnp.int32)
        cnt = jnp.sum((bits >= cand).astype(jnp.float32))
        return jnp.where(cnt >= k, cand, lo)

    tbits = lax.fori_loop(0, 31, body, jnp.int32(0))
    t = lax.bitcast_convert_type(tbits, jnp.float32)
    gt = (bits > tbits).astype(jnp.float32)
    cnt_gt = jnp.sum(gt)
    sum_gt = jnp.sum(floss * gt)
    negA = sum_gt + (k - cnt_gt) * _sl1_of_mag(t)
    delta = jnp.where(needA, negA - S_nz, 0.0)

    @pl.when(p == 0)
    def _():
        acc_ref[...] = jnp.zeros_like(acc_ref)

    acc_ref[...] += _field_block([delta], (8, 128), 0)


def _fallback_call(y_true, y_pred):
    B, C, H, W = y_true.shape
    spec = pl.BlockSpec((1, 1, H, W), lambda p: (p // 4, p % 4, 0, 0))
    return pl.pallas_call(
        _fallback_body,
        grid=(_PLANES,),
        in_specs=[spec, spec],
        out_specs=pl.BlockSpec((8, 128), lambda p: (0, 0)),
        out_shape=jax.ShapeDtypeStruct((8, 128), jnp.float32),
        compiler_params=pltpu.CompilerParams(
            dimension_semantics=("arbitrary",)),
    )(y_true, y_pred)


@jax.jit
def kernel(y_true, y_pred):
    acc, rz3, rp3, st3 = _dense_call(y_true, y_pred)
    neg_rows = _sc_finish(rz3, rp3, st3, y_true, y_pred)
    negB = jnp.sum(neg_rows[:, 0])
    pos_cnt = jnp.maximum(acc[0, 0], 1.0)
    neg_cnt = jnp.maximum(acc[1, 0], 1.0)
    delta = lax.cond(
        acc[3, 0] > 0.5,
        lambda: _fallback_call(y_true, y_pred)[0, 0],
        lambda: jnp.float32(0.0),
    )
    return _NEG_POS * acc[2, 0] / pos_cnt + (negB + delta) / neg_cnt


# speedup vs baseline: 1.2812x; 1.0094x over previous
import functools

import jax
import jax.numpy as jnp
from jax import lax
from jax.experimental import pallas as pl
from jax.experimental.pallas import tpu as pltpu
from jax.experimental.pallas import tpu_sc as plsc

_NEG_POS = 3.0
_H = 512
_W = 512
_N = float(_H * _W)
_PLANES = 64
_SC_NC = 2
_SC_NS = 16


def _sl1_of_mag(x):
    return jnp.where(x < 1.0, 0.5 * x * x, x - 0.5)


def _field_block(fields, shape, axis):
    ii = lax.broadcasted_iota(jnp.int32, shape, axis)
    out = jnp.zeros(shape, jnp.float32)
    for r, f in enumerate(fields):
        out = out + jnp.where(ii == r, f, 0.0)
    return out


_BP = 2
_CP = _BP * 4


def _dense_body(yt_ref, yp_ref, acc_ref, rz_ref, rp_ref, st_ref):
    g = pl.program_id(0)

    yt = yt_ref[...].reshape(_CP * _H, _W)
    yp = yp_ref[...].reshape(_CP * _H, _W)
    ad = jnp.abs(yt - yp)
    sl1 = _sl1_of_mag(ad)
    posb = yt >= 0.5
    posf = posb.astype(jnp.float32)
    z = jnp.logical_or(posb, ad == 0.0).astype(jnp.float32)
    psl1 = sl1 * posf

    rz = jnp.sum(z, axis=1, keepdims=True)
    rp = jnp.sum(psl1, axis=1, keepdims=True)
    rz_ref[...] = rz.reshape(_CP, _H, 1)
    rp_ref[...] = rp.reshape(_CP, _H, 1)

    tot = jnp.zeros((8, 128), jnp.float32)
    for ci in range(_CP):
        sl = slice(ci * _H, (ci + 1) * _H)
        num_pos = jnp.sum(posf[sl])
        sl1_tot = jnp.sum(sl1[sl])
        pos_sl1 = jnp.sum(rp[sl])
        S_nz = sl1_tot - pos_sl1
        nz = _N - jnp.sum(rz[sl])
        k = jnp.minimum(_NEG_POS * num_pos, _N - 1.0)
        needA = jnp.logical_and(k <= nz, k > 0.0).astype(jnp.float32)
        st_ref[ci] = _field_block([num_pos, nz, S_nz, k], (1, 128), 1)
        tot = tot + _field_block([num_pos, k, pos_sl1, needA], (8, 128), 0)

    @pl.when(g == 0)
    def _():
        acc_ref[...] = jnp.zeros_like(acc_ref)

    acc_ref[...] += tot


def _dense_call(y_true, y_pred):
    B, C, H, W = y_true.shape
    spec = pl.BlockSpec((_BP, C, H, W), lambda g: (g, 0, 0, 0))
    return pl.pallas_call(
        _dense_body,
        grid=(_PLANES // _CP,),
        in_specs=[spec, spec],
        out_specs=[
            pl.BlockSpec((8, 128), lambda g: (0, 0)),
            pl.BlockSpec((_CP, H, 1), lambda g: (g, 0, 0)),
            pl.BlockSpec((_CP, H, 1), lambda g: (g, 0, 0)),
            pl.BlockSpec((_CP, 1, 128), lambda g: (g, 0, 0)),
        ],
        out_shape=[
            jax.ShapeDtypeStruct((8, 128), jnp.float32),
            jax.ShapeDtypeStruct((_PLANES, H, 1), jnp.float32),
            jax.ShapeDtypeStruct((_PLANES, H, 1), jnp.float32),
            jax.ShapeDtypeStruct((_PLANES, 1, 128), jnp.float32),
        ],
        compiler_params=pltpu.CompilerParams(
            dimension_semantics=("arbitrary",)),
    )(y_true, y_pred)


def _sc_finish_body(rz_hbm, rp_hbm, st_hbm, yt_hbm, yp_hbm, out_hbm,
                    rzv, rpv, rowt, rowp, stv, outv):
    cid = lax.axis_index("c")
    sid = lax.axis_index("s")
    wid = sid * _SC_NC + cid

    def do_plane(j, carry):
        p = wid * 2 + j
        b = p // 4
        ch = p % 4
        pltpu.sync_copy(st_hbm.at[p], stv)
        pltpu.sync_copy(rz_hbm.at[p], rzv)
        pltpu.sync_copy(rp_hbm.at[p], rpv)
        sv = stv[pl.ds(0, 16)]
        nz = sv[1]
        S_nz = sv[2]
        k = sv[3]
        m = k - nz

        def scan_rows(i, c2):
            cum, rstar, fullsum, ro_r = c2
            v = rzv[pl.ds(i * 16, 16)]
            rpc = rpv[pl.ds(i * 16, 16)]
            inc = plsc.cumsum(v) + cum
            fm = inc <= m
            rstar = rstar + jnp.sum(jnp.where(fm, 1.0, 0.0))
            fullsum = fullsum + jnp.sum(jnp.where(fm, rpc, 0.0))
            ro_r = ro_r + jnp.sum(jnp.where(fm, v, 0.0))
            return jnp.max(inc), rstar, fullsum, ro_r

        _, rstar, fullsum, ro_r = lax.fori_loop(
            0, 32, scan_rows, (0.0, 0.0, 0.0, 0.0))
        r_i = jnp.clip(rstar.astype(jnp.int32), 0, _H - 1)

        pltpu.sync_copy(yt_hbm.at[b, ch, r_i], rowt)
        pltpu.sync_copy(yp_hbm.at[b, ch, r_i], rowp)

        def scan_row(i, c2):
            cumz, partial = c2
            t = rowt[pl.ds(i * 16, 16)]
            q = rowp[pl.ds(i * 16, 16)]
            adv = jnp.abs(t - q)
            posv = t >= 0.5
            zv = jnp.where(jnp.logical_or(posv, adv == 0.0), 1.0, 0.0)
            incz = plsc.cumsum(zv) + cumz
            excl = incz - zv
            sl1v = jnp.where(adv < 1.0, 0.5 * adv * adv, adv - 0.5)
            sel = jnp.logical_and(posv, (ro_r + excl) < m)
            partial = partial + jnp.sum(jnp.where(sel, sl1v, 0.0))
            return jnp.max(incz), partial

        _, partial = lax.fori_loop(0, 32, scan_row, (0.0, 0.0))

        negsum = jnp.where(
            k > 0.0, S_nz + jnp.where(k > nz, fullsum + partial, 0.0), 0.0)
        li = lax.broadcasted_iota(jnp.int32, (16,), 0)
        outv[...] = jnp.where(li == 0, negsum, 0.0)
        pltpu.sync_copy(outv, out_hbm.at[p])
        return carry

    lax.fori_loop(0, _PLANES // (_SC_NC * _SC_NS), do_plane, 0)


def _sc_finish(rz3, rp3, st3, y_true, y_pred):
    mesh = plsc.VectorSubcoreMesh(
        core_axis_name="c", subcore_axis_name="s",
        num_cores=_SC_NC, num_subcores=_SC_NS)
    fn = pl.kernel(
        _sc_finish_body,
        out_type=jax.ShapeDtypeStruct((_PLANES, 16), jnp.float32),
        mesh=mesh,
        compiler_params=pltpu.CompilerParams(needs_layout_passes=False),
        scratch_types=[
            pltpu.VMEM((_H,), jnp.float32),
            pltpu.VMEM((_H,), jnp.float32),
            pltpu.VMEM((_W,), jnp.float32),
            pltpu.VMEM((_W,), jnp.float32),
            pltpu.VMEM((128,), jnp.float32),
            pltpu.VMEM((16,), jnp.float32),
        ],
    )
    return fn(rz3.reshape(_PLANES, _H), rp3.reshape(_PLANES, _H),
              st3.reshape(_PLANES, 128), y_true, y_pred)


def _fallback_body(yt_ref, yp_ref, acc_ref):
    p = pl.program_id(0)

    yt = yt_ref[0, 0]
    yp = yp_ref[0, 0]
    ad = jnp.abs(yt - yp)
    sl1 = _sl1_of_mag(ad)
    posf = (yt >= 0.5).astype(jnp.float32)
    negf = 1.0 - posf
    loss = ad * negf
    floss = sl1 * negf

    num_pos = jnp.sum(posf)
    nz = jnp.sum((loss > 0.0).astype(jnp.float32))
    S_nz = jnp.sum(floss)
    k = jnp.minimum(_NEG_POS * num_pos, _N - 1.0)
    needA = jnp.logical_and(k <= nz, k > 0.0)

    bits = lax.bitcast_convert_type(loss, jnp.int32)

    def body(i, lo):
        cand = lo | (1 << (30 - i)).astype(jnp.int32)
        cnt = jnp.sum((bits >= cand).astype(jnp.float32))
        return jnp.where(cnt >= k, cand, lo)

    tbits = lax.fori_loop(0, 31, body, jnp.int32(0))
    t = lax.bitcast_convert_type(tbits, jnp.float32)
    gt = (bits > tbits).astype(jnp.float32)
    cnt_gt = jnp.sum(gt)
    sum_gt = jnp.sum(floss * gt)
    negA = sum_gt + (k - cnt_gt) * _sl1_of_mag(t)
    delta = jnp.where(needA, negA - S_nz, 0.0)

    @pl.when(p == 0)
    def _():
        acc_ref[...] = jnp.zeros_like(acc_ref)

    acc_ref[...] += _field_block([delta], (8, 128), 0)


def _fallback_call(y_true, y_pred):
    B, C, H, W = y_true.shape
    spec = pl.BlockSpec((1, 1, H, W), lambda p: (p // 4, p % 4, 0, 0))
    return pl.pallas_call(
        _fallback_body,
        grid=(_PLANES,),
        in_specs=[spec, spec],
        out_specs=pl.BlockSpec((8, 128), lambda p: (0, 0)),
        out_shape=jax.ShapeDtypeStruct((8, 128), jnp.float32),
        compiler_params=pltpu.CompilerParams(
            dimension_semantics=("arbitrary",)),
    )(y_true, y_pred)


@jax.jit
def kernel(y_true, y_pred):
    acc, rz3, rp3, st3 = _dense_call(y_true, y_pred)
    neg_rows = _sc_finish(rz3, rp3, st3, y_true, y_pred)
    negB = jnp.sum(neg_rows[:, 0])
    pos_cnt = jnp.maximum(acc[0, 0], 1.0)
    neg_cnt = jnp.maximum(acc[1, 0], 1.0)
    delta = lax.cond(
        acc[3, 0] > 0.5,
        lambda: _fallback_call(y_true, y_pred)[0, 0],
        lambda: jnp.float32(0.0),
    )
    return _NEG_POS * acc[2, 0] / pos_cnt + (negB + delta) / neg_cnt


# speedup vs baseline: 1.3196x; 1.0300x over previous
import functools

import jax
import jax.numpy as jnp
from jax import lax
from jax.experimental import pallas as pl
from jax.experimental.pallas import tpu as pltpu
from jax.experimental.pallas import tpu_sc as plsc

_NEG_POS = 3.0
_H = 512
_W = 512
_N = float(_H * _W)
_PLANES = 64
_SC_NC = 2
_SC_NS = 16


def _sl1_of_mag(x):
    return jnp.where(x < 1.0, 0.5 * x * x, x - 0.5)


def _field_block(fields, shape, axis):
    ii = lax.broadcasted_iota(jnp.int32, shape, axis)
    out = jnp.zeros(shape, jnp.float32)
    for r, f in enumerate(fields):
        out = out + jnp.where(ii == r, f, 0.0)
    return out


_BP = 2
_CP = _BP * 4


def _dense_body(yt_ref, yp_ref, acc_ref, rz_ref, rp_ref, st_ref):
    g = pl.program_id(0)

    yt = yt_ref[...].reshape(_CP * _H, _W)
    yp = yp_ref[...].reshape(_CP * _H, _W)
    ad = jnp.abs(yt - yp)
    sl1 = _sl1_of_mag(ad)
    posb = yt >= 0.5
    posf = posb.astype(jnp.float32)
    z = jnp.logical_or(posb, ad == 0.0).astype(jnp.float32)
    psl1 = sl1 * posf

    rz = jnp.sum(z, axis=1, keepdims=True)
    rp = jnp.sum(psl1, axis=1, keepdims=True)
    rz_ref[...] = rz.reshape(_CP, _H, 1)
    rp_ref[...] = rp.reshape(_CP, _H, 1)

    tot = jnp.zeros((8, 128), jnp.float32)
    for ci in range(_CP):
        sl = slice(ci * _H, (ci + 1) * _H)
        num_pos = jnp.sum(posf[sl])
        sl1_tot = jnp.sum(sl1[sl])
        pos_sl1 = jnp.sum(rp[sl])
        S_nz = sl1_tot - pos_sl1
        nz = _N - jnp.sum(rz[sl])
        k = jnp.minimum(_NEG_POS * num_pos, _N - 1.0)
        needA = jnp.logical_and(k <= nz, k > 0.0).astype(jnp.float32)
        st_ref[ci] = _field_block([num_pos, nz, S_nz, k], (1, 128), 1)
        tot = tot + _field_block([num_pos, k, pos_sl1, needA], (8, 128), 0)

    @pl.when(g == 0)
    def _():
        acc_ref[...] = jnp.zeros_like(acc_ref)

    acc_ref[...] += tot


def _dense_call(y_true, y_pred):
    B, C, H, W = y_true.shape
    spec = pl.BlockSpec((_BP, C, H, W), lambda g: (g, 0, 0, 0))
    return pl.pallas_call(
        _dense_body,
        grid=(_PLANES // _CP,),
        in_specs=[spec, spec],
        out_specs=[
            pl.BlockSpec((8, 128), lambda g: (0, 0)),
            pl.BlockSpec((_CP, H, 1), lambda g: (g, 0, 0)),
            pl.BlockSpec((_CP, H, 1), lambda g: (g, 0, 0)),
            pl.BlockSpec((_CP, 1, 128), lambda g: (g, 0, 0)),
        ],
        out_shape=[
            jax.ShapeDtypeStruct((8, 128), jnp.float32),
            jax.ShapeDtypeStruct((_PLANES, H, 1), jnp.float32),
            jax.ShapeDtypeStruct((_PLANES, H, 1), jnp.float32),
            jax.ShapeDtypeStruct((_PLANES, 1, 128), jnp.float32),
        ],
        compiler_params=pltpu.CompilerParams(
            dimension_semantics=("arbitrary",)),
    )(y_true, y_pred)


def _sc_finish_body(rz_hbm, rp_hbm, st_hbm, yt_hbm, yp_hbm, out_hbm,
                    rzv, rpv, stv, rowt, rowp, outv, sem):
    cid = lax.axis_index("c")
    sid = lax.axis_index("s")
    wid = sid * _SC_NC + cid
    p0 = wid * 2

    copies = []
    for j in range(2):
        copies.append(pltpu.async_copy(st_hbm.at[p0 + j], stv.at[j], sem))
        copies.append(pltpu.async_copy(rz_hbm.at[p0 + j], rzv.at[j], sem))
        copies.append(pltpu.async_copy(rp_hbm.at[p0 + j], rpv.at[j], sem))
    for cp in copies:
        cp.wait()

    total = 0.0
    for j in range(2):
        p = p0 + j
        b = p // 4
        ch = p % 4
        sv = stv[j, pl.ds(0, 16)]
        nz = sv[1]
        S_nz = sv[2]
        k = sv[3]
        m = k - nz

        def scan_rows(i, c2):
            cum, rstar, fullsum, ro_r = c2
            v = rzv[j, pl.ds(i * 16, 16)]
            rpc = rpv[j, pl.ds(i * 16, 16)]
            inc = plsc.cumsum(v) + cum
            fm = inc <= m
            rstar = rstar + jnp.sum(jnp.where(fm, 1.0, 0.0))
            fullsum = fullsum + jnp.sum(jnp.where(fm, rpc, 0.0))
            ro_r = ro_r + jnp.sum(jnp.where(fm, v, 0.0))
            return jnp.max(inc), rstar, fullsum, ro_r

        _, rstar, fullsum, ro_r = lax.fori_loop(
            0, 32, scan_rows, (0.0, 0.0, 0.0, 0.0))
        r_i = jnp.clip(rstar.astype(jnp.int32), 0, _H - 1)

        c1 = pltpu.async_copy(yt_hbm.at[b, ch, r_i], rowt, sem)
        c2_ = pltpu.async_copy(yp_hbm.at[b, ch, r_i], rowp, sem)
        c1.wait()
        c2_.wait()

        def scan_row(i, c2):
            cumz, partial = c2
            t = rowt[pl.ds(i * 16, 16)]
            q = rowp[pl.ds(i * 16, 16)]
            adv = jnp.abs(t - q)
            posv = t >= 0.5
            zv = jnp.where(jnp.logical_or(posv, adv == 0.0), 1.0, 0.0)
            incz = plsc.cumsum(zv) + cumz
            excl = incz - zv
            sl1v = jnp.where(adv < 1.0, 0.5 * adv * adv, adv - 0.5)
            sel = jnp.logical_and(posv, (ro_r + excl) < m)
            partial = partial + jnp.sum(jnp.where(sel, sl1v, 0.0))
            return jnp.max(incz), partial

        _, partial = lax.fori_loop(0, 32, scan_row, (0.0, 0.0))

        total = total + jnp.where(
            k > 0.0, S_nz + jnp.where(k > nz, fullsum + partial, 0.0), 0.0)

    li = lax.broadcasted_iota(jnp.int32, (16,), 0)
    outv[...] = jnp.where(li == 0, total, 0.0)
    pltpu.sync_copy(outv, out_hbm.at[wid])


def _sc_finish(rz3, rp3, st3, y_true, y_pred):
    mesh = plsc.VectorSubcoreMesh(
        core_axis_name="c", subcore_axis_name="s",
        num_cores=_SC_NC, num_subcores=_SC_NS)
    fn = pl.kernel(
        _sc_finish_body,
        out_type=jax.ShapeDtypeStruct((_SC_NC * _SC_NS, 16), jnp.float32),
        mesh=mesh,
        compiler_params=pltpu.CompilerParams(needs_layout_passes=False),
        scratch_types=[
            pltpu.VMEM((2, _H), jnp.float32),
            pltpu.VMEM((2, _H), jnp.float32),
            pltpu.VMEM((2, 128), jnp.float32),
            pltpu.VMEM((_W,), jnp.float32),
            pltpu.VMEM((_W,), jnp.float32),
            pltpu.VMEM((16,), jnp.float32),
            pltpu.SemaphoreType.DMA,
        ],
    )
    return fn(rz3.reshape(_PLANES, _H), rp3.reshape(_PLANES, _H),
              st3.reshape(_PLANES, 128), y_true, y_pred)


def _fallback_body(yt_ref, yp_ref, acc_ref):
    p = pl.program_id(0)

    yt = yt_ref[0, 0]
    yp = yp_ref[0, 0]
    ad = jnp.abs(yt - yp)
    sl1 = _sl1_of_mag(ad)
    posf = (yt >= 0.5).astype(jnp.float32)
    negf = 1.0 - posf
    loss = ad * negf
    floss = sl1 * negf

    num_pos = jnp.sum(posf)
    nz = jnp.sum((loss > 0.0).astype(jnp.float32))
    S_nz = jnp.sum(floss)
    k = jnp.minimum(_NEG_POS * num_pos, _N - 1.0)
    needA = jnp.logical_and(k <= nz, k > 0.0)

    bits = lax.bitcast_convert_type(loss, jnp.int32)

    def body(i, lo):
        cand = lo | (1 << (30 - i)).astype(jnp.int32)
        cnt = jnp.sum((bits >= cand).astype(jnp.float32))
        return jnp.where(cnt >= k, cand, lo)

    tbits = lax.fori_loop(0, 31, body, jnp.int32(0))
    t = lax.bitcast_convert_type(tbits, jnp.float32)
    gt = (bits > tbits).astype(jnp.float32)
    cnt_gt = jnp.sum(gt)
    sum_gt = jnp.sum(floss * gt)
    negA = sum_gt + (k - cnt_gt) * _sl1_of_mag(t)
    delta = jnp.where(needA, negA - S_nz, 0.0)

    @pl.when(p == 0)
    def _():
        acc_ref[...] = jnp.zeros_like(acc_ref)

    acc_ref[...] += _field_block([delta], (8, 128), 0)


def _fallback_call(y_true, y_pred):
    B, C, H, W = y_true.shape
    spec = pl.BlockSpec((1, 1, H, W), lambda p: (p // 4, p % 4, 0, 0))
    return pl.pallas_call(
        _fallback_body,
        grid=(_PLANES,),
        in_specs=[spec, spec],
        out_specs=pl.BlockSpec((8, 128), lambda p: (0, 0)),
        out_shape=jax.ShapeDtypeStruct((8, 128), jnp.float32),
        compiler_params=pltpu.CompilerParams(
            dimension_semantics=("arbitrary",)),
    )(y_true, y_pred)


@jax.jit
def kernel(y_true, y_pred):
    acc, rz3, rp3, st3 = _dense_call(y_true, y_pred)
    neg_rows = _sc_finish(rz3, rp3, st3, y_true, y_pred)
    negB = jnp.sum(neg_rows[:, 0])
    pos_cnt = jnp.maximum(acc[0, 0], 1.0)
    neg_cnt = jnp.maximum(acc[1, 0], 1.0)
    delta = lax.cond(
        acc[3, 0] > 0.5,
        lambda: _fallback_call(y_true, y_pred)[0, 0],
        lambda: jnp.float32(0.0),
    )
    return _NEG_POS * acc[2, 0] / pos_cnt + (negB + delta) / neg_cnt


# speedup vs baseline: 1.3209x; 1.0010x over previous
import jax
import jax.numpy as jnp
from jax import lax
from jax.experimental import pallas as pl
from jax.experimental.pallas import tpu as pltpu
from jax.experimental.pallas import tpu_sc as plsc

_NEG_POS = 3.0
_H = 512
_W = 512
_N = float(_H * _W)
_PLANES = 64
_SC_NC = 2
_SC_NS = 16


def _sl1_of_mag(x):
    return jnp.where(x < 1.0, 0.5 * x * x, x - 0.5)


def _field_block(fields, shape, axis):
    ii = lax.broadcasted_iota(jnp.int32, shape, axis)
    out = jnp.zeros(shape, jnp.float32)
    for r, f in enumerate(fields):
        out = out + jnp.where(ii == r, f, 0.0)
    return out


_BP = 2
_CP = _BP * 4


def _dense_body(yt_ref, yp_ref, acc_ref, rz_ref, rp_ref, st_ref):
    g = pl.program_id(0)

    yt = yt_ref[...].reshape(_CP * _H, _W)
    yp = yp_ref[...].reshape(_CP * _H, _W)
    ad = jnp.abs(yt - yp)
    sl1 = _sl1_of_mag(ad)
    posb = yt >= 0.5
    posf = posb.astype(jnp.float32)
    z = jnp.logical_or(posb, ad == 0.0).astype(jnp.float32)
    psl1 = sl1 * posf

    rz = jnp.sum(z, axis=1, keepdims=True)
    rp = jnp.sum(psl1, axis=1, keepdims=True)
    rz_ref[...] = rz.reshape(_CP, _H, 1)
    rp_ref[...] = rp.reshape(_CP, _H, 1)

    tot = jnp.zeros((8, 128), jnp.float32)
    for ci in range(_CP):
        sl = slice(ci * _H, (ci + 1) * _H)
        num_pos = jnp.sum(posf[sl])
        sl1_tot = jnp.sum(sl1[sl])
        pos_sl1 = jnp.sum(rp[sl])
        S_nz = sl1_tot - pos_sl1
        nz = _N - jnp.sum(rz[sl])
        k = jnp.minimum(_NEG_POS * num_pos, _N - 1.0)
        needA = jnp.logical_and(k <= nz, k > 0.0).astype(jnp.float32)
        st_ref[ci] = _field_block([num_pos, nz, S_nz, k], (1, 128), 1)
        tot = tot + _field_block([num_pos, k, pos_sl1, needA], (8, 128), 0)

    @pl.when(g == 0)
    def _():
        acc_ref[...] = jnp.zeros_like(acc_ref)

    acc_ref[...] += tot


def _dense_call(y_true, y_pred):
    B, C, H, W = y_true.shape
    spec = pl.BlockSpec((_BP, C, H, W), lambda g: (g, 0, 0, 0))
    return pl.pallas_call(
        _dense_body,
        grid=(_PLANES // _CP,),
        in_specs=[spec, spec],
        out_specs=[
            pl.BlockSpec((8, 128), lambda g: (0, 0)),
            pl.BlockSpec((_CP, H, 1), lambda g: (g, 0, 0)),
            pl.BlockSpec((_CP, H, 1), lambda g: (g, 0, 0)),
            pl.BlockSpec((_CP, 1, 128), lambda g: (g, 0, 0)),
        ],
        out_shape=[
            jax.ShapeDtypeStruct((8, 128), jnp.float32),
            jax.ShapeDtypeStruct((_PLANES, H, 1), jnp.float32),
            jax.ShapeDtypeStruct((_PLANES, H, 1), jnp.float32),
            jax.ShapeDtypeStruct((_PLANES, 1, 128), jnp.float32),
        ],
        compiler_params=pltpu.CompilerParams(
            dimension_semantics=("arbitrary",)),
    )(y_true, y_pred)


def _sc_finish_body(rz_hbm, rp_hbm, st_hbm, yt_hbm, yp_hbm, out_hbm,
                    rzv, rpv, stv, rowt, rowp, outv, sem):
    cid = lax.axis_index("c")
    sid = lax.axis_index("s")
    wid = sid * _SC_NC + cid
    p0 = wid * 2

    copies = []
    for j in range(2):
        copies.append(pltpu.async_copy(st_hbm.at[p0 + j], stv.at[j], sem))
        copies.append(pltpu.async_copy(rz_hbm.at[p0 + j], rzv.at[j], sem))
        copies.append(pltpu.async_copy(rp_hbm.at[p0 + j], rpv.at[j], sem))
    for cp in copies:
        cp.wait()

    total = 0.0
    for j in range(2):
        p = p0 + j
        b = p // 4
        ch = p % 4
        sv = stv[j, pl.ds(0, 16)]
        nz = sv[1]
        S_nz = sv[2]
        k = sv[3]
        m = k - nz

        def scan_rows(i, c2):
            cum, rstar, fullsum, ro_r = c2
            v = rzv[j, pl.ds(i * 16, 16)]
            rpc = rpv[j, pl.ds(i * 16, 16)]
            inc = plsc.cumsum(v) + cum
            fm = inc <= m
            rstar = rstar + jnp.sum(jnp.where(fm, 1.0, 0.0))
            fullsum = fullsum + jnp.sum(jnp.where(fm, rpc, 0.0))
            ro_r = ro_r + jnp.sum(jnp.where(fm, v, 0.0))
            return jnp.max(inc), rstar, fullsum, ro_r

        _, rstar, fullsum, ro_r = lax.fori_loop(
            0, 32, scan_rows, (0.0, 0.0, 0.0, 0.0))
        r_i = jnp.clip(rstar.astype(jnp.int32), 0, _H - 1)

        c1 = pltpu.async_copy(yt_hbm.at[b, ch, r_i], rowt, sem)
        c2_ = pltpu.async_copy(yp_hbm.at[b, ch, r_i], rowp, sem)
        c1.wait()
        c2_.wait()

        def scan_row(i, c2):
            cumz, partial = c2
            t = rowt[pl.ds(i * 16, 16)]
            q = rowp[pl.ds(i * 16, 16)]
            adv = jnp.abs(t - q)
            posv = t >= 0.5
            zv = jnp.where(jnp.logical_or(posv, adv == 0.0), 1.0, 0.0)
            incz = plsc.cumsum(zv) + cumz
            excl = incz - zv
            sl1v = jnp.where(adv < 1.0, 0.5 * adv * adv, adv - 0.5)
            sel = jnp.logical_and(posv, (ro_r + excl) < m)
            partial = partial + jnp.sum(jnp.where(sel, sl1v, 0.0))
            return jnp.max(incz), partial

        _, partial = lax.fori_loop(0, 32, scan_row, (0.0, 0.0))

        total = total + jnp.where(
            k > 0.0, S_nz + jnp.where(k > nz, fullsum + partial, 0.0), 0.0)

    li = lax.broadcasted_iota(jnp.int32, (16,), 0)
    outv[...] = jnp.where(li == 0, total, 0.0)
    pltpu.sync_copy(outv, out_hbm.at[wid])


def _sc_finish(rz3, rp3, st3, y_true, y_pred):
    mesh = plsc.VectorSubcoreMesh(
        core_axis_name="c", subcore_axis_name="s",
        num_cores=_SC_NC, num_subcores=_SC_NS)
    fn = pl.kernel(
        _sc_finish_body,
        out_type=jax.ShapeDtypeStruct((_SC_NC * _SC_NS, 16), jnp.float32),
        mesh=mesh,
        compiler_params=pltpu.CompilerParams(needs_layout_passes=False),
        scratch_types=[
            pltpu.VMEM((2, _H), jnp.float32),
            pltpu.VMEM((2, _H), jnp.float32),
            pltpu.VMEM((2, 128), jnp.float32),
            pltpu.VMEM((_W,), jnp.float32),
            pltpu.VMEM((_W,), jnp.float32),
            pltpu.VMEM((16,), jnp.float32),
            pltpu.SemaphoreType.DMA,
        ],
    )
    return fn(rz3.reshape(_PLANES, _H), rp3.reshape(_PLANES, _H),
              st3.reshape(_PLANES, 128), y_true, y_pred)


def _fallback_body(yt_ref, yp_ref, acc_ref):
    p = pl.program_id(0)

    yt = yt_ref[0, 0]
    yp = yp_ref[0, 0]
    ad = jnp.abs(yt - yp)
    sl1 = _sl1_of_mag(ad)
    posf = (yt >= 0.5).astype(jnp.float32)
    negf = 1.0 - posf
    loss = ad * negf
    floss = sl1 * negf

    num_pos = jnp.sum(posf)
    nz = jnp.sum((loss > 0.0).astype(jnp.float32))
    S_nz = jnp.sum(floss)
    k = jnp.minimum(_NEG_POS * num_pos, _N - 1.0)
    needA = jnp.logical_and(k <= nz, k > 0.0)

    bits = lax.bitcast_convert_type(loss, jnp.int32)

    def body(i, lo):
        cand = lo | (1 << (30 - i)).astype(jnp.int32)
        cnt = jnp.sum((bits >= cand).astype(jnp.float32))
        return jnp.where(cnt >= k, cand, lo)

    tbits = lax.fori_loop(0, 31, body, jnp.int32(0))
    t = lax.bitcast_convert_type(tbits, jnp.float32)
    gt = (bits > tbits).astype(jnp.float32)
    cnt_gt = jnp.sum(gt)
    sum_gt = jnp.sum(floss * gt)
    negA = sum_gt + (k - cnt_gt) * _sl1_of_mag(t)
    delta = jnp.where(needA, negA - S_nz, 0.0)

    @pl.when(p == 0)
    def _():
        acc_ref[...] = jnp.zeros_like(acc_ref)

    acc_ref[...] += _field_block([delta], (8, 128), 0)


def _fallback_call(y_true, y_pred):
    B, C, H, W = y_true.shape
    spec = pl.BlockSpec((1, 1, H, W), lambda p: (p // 4, p % 4, 0, 0))
    return pl.pallas_call(
        _fallback_body,
        grid=(_PLANES,),
        in_specs=[spec, spec],
        out_specs=pl.BlockSpec((8, 128), lambda p: (0, 0)),
        out_shape=jax.ShapeDtypeStruct((8, 128), jnp.float32),
        compiler_params=pltpu.CompilerParams(
            dimension_semantics=("arbitrary",)),
    )(y_true, y_pred)


@jax.jit
def kernel(y_true, y_pred):
    acc, rz3, rp3, st3 = _dense_call(y_true, y_pred)
    neg_rows = _sc_finish(rz3, rp3, st3, y_true, y_pred)
    negB = jnp.sum(neg_rows[:, 0])
    pos_cnt = jnp.maximum(acc[0, 0], 1.0)
    neg_cnt = jnp.maximum(acc[1, 0], 1.0)
    delta = lax.cond(
        acc[3, 0] > 0.5,
        lambda: _fallback_call(y_true, y_pred)[0, 0],
        lambda: jnp.float32(0.0),
    )
    return _NEG_POS * acc[2, 0] / pos_cnt + (negB + delta) / neg_cnt
